# baseline (device time: 166734 ns/iter reference)
import jax
import jax.numpy as jnp
from jax import lax
from jax.experimental import pallas as pl
from jax.experimental.pallas import tpu as pltpu

N_DEV = 4


def kernel(O, Wo):
    B, S, H, D = O.shape
    HD = H * D
    N = Wo.shape[1]
    s_chunk = S // N_DEV
    rows = B * s_chunk

    O_r = (
        O.reshape(B, N_DEV, s_chunk, HD)
        .transpose(1, 0, 2, 3)
        .reshape(N_DEV * rows, HD)
        .astype(jnp.bfloat16)
    )
    Wo_b = Wo.astype(jnp.bfloat16)

    def body(o_ref, wo_ref, out_ref, comm_ref, send_sems, recv_sems):
        my = lax.axis_index("i")
        left = lax.rem(my + (N_DEV - 1), N_DEV)
        right = lax.rem(my + 1, N_DEV)

        barrier_sem = pltpu.get_barrier_semaphore()
        for nbr in (left, right):
            pl.semaphore_signal(
                barrier_sem, inc=1,
                device_id=(nbr,), device_id_type=pl.DeviceIdType.MESH,
            )
        pl.semaphore_wait(barrier_sem, 2)

        def local_partial(c):
            x = o_ref[pl.ds(c * rows, rows), :]
            return jnp.dot(x, wo_ref[:, :], preferred_element_type=jnp.float32)

        comm_ref[0, :, :] = local_partial(
            lax.rem(my + (N_DEV - 1), N_DEV)
        ).astype(jnp.bfloat16)

        for h in range(N_DEV - 1):
            rdma = pltpu.make_async_remote_copy(
                src_ref=comm_ref.at[h],
                dst_ref=comm_ref.at[h + 1],
                send_sem=send_sems.at[h],
                recv_sem=recv_sems.at[h],
                device_id=(right,),
                device_id_type=pl.DeviceIdType.MESH,
            )
            rdma.start()
            nxt = local_partial(lax.rem(my + (N_DEV - 2) - h, N_DEV))
            rdma.wait()
            if h < N_DEV - 2:
                comm_ref[h + 1, :, :] = (
                    comm_ref[h + 1, :, :].astype(jnp.float32) + nxt
                ).astype(jnp.bfloat16)
            else:
                out_ref[:, :] = comm_ref[h + 1, :, :].astype(jnp.float32) + nxt

    out = pl.pallas_call(
        body,
        out_shape=jax.ShapeDtypeStruct((rows, N), jnp.float32),
        in_specs=[
            pl.BlockSpec(memory_space=pltpu.VMEM),
            pl.BlockSpec(memory_space=pltpu.VMEM),
        ],
        out_specs=pl.BlockSpec(memory_space=pltpu.VMEM),
        scratch_shapes=[
            pltpu.VMEM((N_DEV, rows, N), jnp.bfloat16),
            pltpu.SemaphoreType.DMA((N_DEV - 1,)),
            pltpu.SemaphoreType.DMA((N_DEV - 1,)),
        ],
        compiler_params=pltpu.CompilerParams(collective_id=0),
    )(O_r, Wo_b)

    return out.reshape(B, s_chunk, N)


# device time: 99375 ns/iter; 1.6778x vs baseline; 1.6778x over previous
import jax
import jax.numpy as jnp
from jax import lax
from jax.experimental import pallas as pl
from jax.experimental.pallas import tpu as pltpu

N_DEV = 4


def kernel(O, Wo):
    B, S, H, D = O.shape
    HD = H * D
    N = Wo.shape[1]
    Nh = N // 2
    s_chunk = S // N_DEV
    rows = B * s_chunk

    O_r = (
        O.reshape(B, N_DEV, s_chunk, HD)
        .transpose(1, 0, 2, 3)
        .reshape(N_DEV * rows, HD)
        .astype(jnp.bfloat16)
    )
    Wo_b = Wo.astype(jnp.bfloat16)

    def body(o_ref, wo_ref, out_ref,
             comm_r, comm_l, send_r, recv_r, send_l, recv_l):
        my = lax.axis_index("i")
        left = lax.rem(my + (N_DEV - 1), N_DEV)
        right = lax.rem(my + 1, N_DEV)

        barrier_sem = pltpu.get_barrier_semaphore()
        for nbr in (left, right):
            pl.semaphore_signal(
                barrier_sem, inc=1,
                device_id=(nbr,), device_id_type=pl.DeviceIdType.MESH,
            )
        pl.semaphore_wait(barrier_sem, 2)

        def partial(c, col0):
            x = o_ref[pl.ds(c * rows, rows), :]
            w = wo_ref[:, col0:col0 + Nh]
            return jnp.dot(x, w, preferred_element_type=jnp.float32)

        comm_r[0, :, :] = partial(lax.rem(my + 3, N_DEV), 0).astype(jnp.bfloat16)
        comm_l[0, :, :] = partial(lax.rem(my + 1, N_DEV), Nh).astype(jnp.bfloat16)

        for h in range(N_DEV - 1):
            rdma_r = pltpu.make_async_remote_copy(
                src_ref=comm_r.at[h], dst_ref=comm_r.at[h + 1],
                send_sem=send_r.at[h], recv_sem=recv_r.at[h],
                device_id=(right,), device_id_type=pl.DeviceIdType.MESH,
            )
            rdma_l = pltpu.make_async_remote_copy(
                src_ref=comm_l.at[h], dst_ref=comm_l.at[h + 1],
                send_sem=send_l.at[h], recv_sem=recv_l.at[h],
                device_id=(left,), device_id_type=pl.DeviceIdType.MESH,
            )
            rdma_r.start()
            rdma_l.start()
            nxt_r = partial(lax.rem(my + 2 - h, N_DEV), 0)
            nxt_l = partial(lax.rem(my + 2 + h, N_DEV), Nh)
            rdma_r.wait()
            rdma_l.wait()
            if h < N_DEV - 2:
                comm_r[h + 1, :, :] = (
                    comm_r[h + 1, :, :].astype(jnp.float32) + nxt_r
                ).astype(jnp.bfloat16)
                comm_l[h + 1, :, :] = (
                    comm_l[h + 1, :, :].astype(jnp.float32) + nxt_l
                ).astype(jnp.bfloat16)
            else:
                out_ref[:, 0:Nh] = comm_r[h + 1, :, :].astype(jnp.float32) + nxt_r
                out_ref[:, Nh:N] = comm_l[h + 1, :, :].astype(jnp.float32) + nxt_l

    out = pl.pallas_call(
        body,
        out_shape=jax.ShapeDtypeStruct((rows, N), jnp.float32),
        in_specs=[
            pl.BlockSpec(memory_space=pltpu.VMEM),
            pl.BlockSpec(memory_space=pltpu.VMEM),
        ],
        out_specs=pl.BlockSpec(memory_space=pltpu.VMEM),
        scratch_shapes=[
            pltpu.VMEM((N_DEV, rows, Nh), jnp.bfloat16),
            pltpu.VMEM((N_DEV, rows, Nh), jnp.bfloat16),
            pltpu.SemaphoreType.DMA((N_DEV - 1,)),
            pltpu.SemaphoreType.DMA((N_DEV - 1,)),
            pltpu.SemaphoreType.DMA((N_DEV - 1,)),
            pltpu.SemaphoreType.DMA((N_DEV - 1,)),
        ],
        compiler_params=pltpu.CompilerParams(collective_id=0),
    )(O_r, Wo_b)

    return out.reshape(B, s_chunk, N)


# device time: 91115 ns/iter; 1.8299x vs baseline; 1.0907x over previous
import jax
import jax.numpy as jnp
from jax import lax
from jax.experimental import pallas as pl
from jax.experimental.pallas import tpu as pltpu

N_DEV = 4
SEG = 4


def kernel(O, Wo):
    B, S, H, D = O.shape
    HD = H * D
    N = Wo.shape[1]
    Nh = N // 2
    s_chunk = S // N_DEV
    rows = B * s_chunk
    seg_rows = rows // SEG

    O_r = (
        O.reshape(B, N_DEV, s_chunk, HD)
        .transpose(1, 0, 2, 3)
        .reshape(N_DEV * rows, HD)
        .astype(jnp.bfloat16)
    )
    Wo_b = Wo.astype(jnp.bfloat16)

    def body(o_ref, wo_ref, out_ref,
             comm_r, comm_l, send_r, recv_r, send_l, recv_l):
        my = lax.axis_index("i")
        left = lax.rem(my + (N_DEV - 1), N_DEV)
        right = lax.rem(my + 1, N_DEV)

        barrier_sem = pltpu.get_barrier_semaphore()
        for nbr in (left, right):
            pl.semaphore_signal(
                barrier_sem, inc=1,
                device_id=(nbr,), device_id_type=pl.DeviceIdType.MESH,
            )
        pl.semaphore_wait(barrier_sem, 2)

        def partial(c, s, col0):
            x = o_ref[pl.ds(c * rows + s * seg_rows, seg_rows), :]
            w = wo_ref[:, col0:col0 + Nh]
            return jnp.dot(x, w, preferred_element_type=jnp.float32)

        def make_rdma(comm, send, recv, h, s, dev):
            return pltpu.make_async_remote_copy(
                src_ref=comm.at[h, s], dst_ref=comm.at[h + 1, s],
                send_sem=send.at[h, s], recv_sem=recv.at[h, s],
                device_id=(dev,), device_id_type=pl.DeviceIdType.MESH,
            )

        in_flight = []

        def start(comm, send, recv, h, s, dev):
            rdma = make_rdma(comm, send, recv, h, s, dev)
            rdma.start()
            in_flight.append(rdma)

        rcv_r = [lax.rem(my + 2 - h, N_DEV) for h in range(N_DEV - 1)]
        rcv_l = [lax.rem(my + 2 + h, N_DEV) for h in range(N_DEV - 1)]

        c_r0 = lax.rem(my + 3, N_DEV)
        c_l0 = lax.rem(my + 1, N_DEV)
        for s in range(SEG):
            comm_r[0, s] = partial(c_r0, s, 0).astype(jnp.bfloat16)
            start(comm_r, send_r, recv_r, 0, s, right)
            comm_l[0, s] = partial(c_l0, s, Nh).astype(jnp.bfloat16)
            start(comm_l, send_l, recv_l, 0, s, left)

        for h in range(N_DEV - 2):
            for s in range(SEG):
                nxt_r = partial(rcv_r[h], s, 0)
                make_rdma(comm_r, send_r, recv_r, h, s, right).wait_recv()
                comm_r[h + 1, s] = (
                    comm_r[h + 1, s].astype(jnp.float32) + nxt_r
                ).astype(jnp.bfloat16)
                start(comm_r, send_r, recv_r, h + 1, s, right)

                nxt_l = partial(rcv_l[h], s, Nh)
                make_rdma(comm_l, send_l, recv_l, h, s, left).wait_recv()
                comm_l[h + 1, s] = (
                    comm_l[h + 1, s].astype(jnp.float32) + nxt_l
                ).astype(jnp.bfloat16)
                start(comm_l, send_l, recv_l, h + 1, s, left)

        hf = N_DEV - 2
        for s in range(SEG):
            nxt_r = partial(my, s, 0)
            make_rdma(comm_r, send_r, recv_r, hf, s, right).wait_recv()
            out_ref[pl.ds(s * seg_rows, seg_rows), 0:Nh] = (
                comm_r[hf + 1, s].astype(jnp.float32) + nxt_r
            )
            nxt_l = partial(my, s, Nh)
            make_rdma(comm_l, send_l, recv_l, hf, s, left).wait_recv()
            out_ref[pl.ds(s * seg_rows, seg_rows), Nh:N] = (
                comm_l[hf + 1, s].astype(jnp.float32) + nxt_l
            )

        for rdma in in_flight:
            rdma.wait_send()

    out = pl.pallas_call(
        body,
        out_shape=jax.ShapeDtypeStruct((rows, N), jnp.float32),
        in_specs=[
            pl.BlockSpec(memory_space=pltpu.VMEM),
            pl.BlockSpec(memory_space=pltpu.VMEM),
        ],
        out_specs=pl.BlockSpec(memory_space=pltpu.VMEM),
        scratch_shapes=[
            pltpu.VMEM((N_DEV, SEG, seg_rows, Nh), jnp.bfloat16),
            pltpu.VMEM((N_DEV, SEG, seg_rows, Nh), jnp.bfloat16),
            pltpu.SemaphoreType.DMA((N_DEV - 1, SEG)),
            pltpu.SemaphoreType.DMA((N_DEV - 1, SEG)),
            pltpu.SemaphoreType.DMA((N_DEV - 1, SEG)),
            pltpu.SemaphoreType.DMA((N_DEV - 1, SEG)),
        ],
        compiler_params=pltpu.CompilerParams(collective_id=0),
    )(O_r, Wo_b)

    return out.reshape(B, s_chunk, N)


# device time: 90417 ns/iter; 1.8441x vs baseline; 1.0077x over previous
import jax
import jax.numpy as jnp
from jax import lax
from jax.experimental import pallas as pl
from jax.experimental.pallas import tpu as pltpu

N_DEV = 4


def kernel(O, Wo):
    B, S, H, D = O.shape
    HD = H * D
    N = Wo.shape[1]
    Nh = N // 2
    s_chunk = S // N_DEV
    SEG = B

    O_f = O.reshape(B * S, HD)

    def body(o_ref, wo_ref, out_ref,
             wo_bf, comm_r, comm_l, send_r, recv_r, send_l, recv_l):
        my = lax.axis_index("i")
        left = lax.rem(my + (N_DEV - 1), N_DEV)
        right = lax.rem(my + 1, N_DEV)

        barrier_sem = pltpu.get_barrier_semaphore()
        for nbr in (left, right):
            pl.semaphore_signal(
                barrier_sem, inc=1,
                device_id=(nbr,), device_id_type=pl.DeviceIdType.MESH,
            )
        wo_bf[...] = wo_ref[...].astype(jnp.bfloat16)
        pl.semaphore_wait(barrier_sem, 2)

        def partial(c, b, col0):
            x = o_ref[pl.ds(b * S + c * s_chunk, s_chunk), :]
            w = wo_bf[:, col0:col0 + Nh]
            return jnp.dot(
                x.astype(jnp.bfloat16), w, preferred_element_type=jnp.float32
            )

        def make_rdma(comm, send, recv, h, s, dev):
            return pltpu.make_async_remote_copy(
                src_ref=comm.at[h, s], dst_ref=comm.at[h + 1, s],
                send_sem=send.at[h, s], recv_sem=recv.at[h, s],
                device_id=(dev,), device_id_type=pl.DeviceIdType.MESH,
            )

        in_flight = []

        def start(comm, send, recv, h, s, dev):
            rdma = make_rdma(comm, send, recv, h, s, dev)
            rdma.start()
            in_flight.append(rdma)

        rcv_r = [lax.rem(my + 2 - h, N_DEV) for h in range(N_DEV - 1)]
        rcv_l = [lax.rem(my + 2 + h, N_DEV) for h in range(N_DEV - 1)]

        c_r0 = lax.rem(my + 3, N_DEV)
        c_l0 = lax.rem(my + 1, N_DEV)
        for s in range(SEG):
            comm_r[0, s] = partial(c_r0, s, 0).astype(jnp.bfloat16)
            start(comm_r, send_r, recv_r, 0, s, right)
            comm_l[0, s] = partial(c_l0, s, Nh).astype(jnp.bfloat16)
            start(comm_l, send_l, recv_l, 0, s, left)

        for h in range(N_DEV - 2):
            for s in range(SEG):
                nxt_r = partial(rcv_r[h], s, 0)
                make_rdma(comm_r, send_r, recv_r, h, s, right).wait_recv()
                comm_r[h + 1, s] = (
                    comm_r[h + 1, s].astype(jnp.float32) + nxt_r
                ).astype(jnp.bfloat16)
                start(comm_r, send_r, recv_r, h + 1, s, right)

                nxt_l = partial(rcv_l[h], s, Nh)
                make_rdma(comm_l, send_l, recv_l, h, s, left).wait_recv()
                comm_l[h + 1, s] = (
                    comm_l[h + 1, s].astype(jnp.float32) + nxt_l
                ).astype(jnp.bfloat16)
                start(comm_l, send_l, recv_l, h + 1, s, left)

        hf = N_DEV - 2
        for s in range(SEG):
            nxt_r = partial(my, s, 0)
            make_rdma(comm_r, send_r, recv_r, hf, s, right).wait_recv()
            out_ref[pl.ds(s * s_chunk, s_chunk), 0:Nh] = (
                comm_r[hf + 1, s].astype(jnp.float32) + nxt_r
            )
            nxt_l = partial(my, s, Nh)
            make_rdma(comm_l, send_l, recv_l, hf, s, left).wait_recv()
            out_ref[pl.ds(s * s_chunk, s_chunk), Nh:N] = (
                comm_l[hf + 1, s].astype(jnp.float32) + nxt_l
            )

        for rdma in in_flight:
            rdma.wait_send()

    out = pl.pallas_call(
        body,
        out_shape=jax.ShapeDtypeStruct((B * s_chunk, N), jnp.float32),
        in_specs=[
            pl.BlockSpec(memory_space=pltpu.VMEM),
            pl.BlockSpec(memory_space=pltpu.VMEM),
        ],
        out_specs=pl.BlockSpec(memory_space=pltpu.VMEM),
        scratch_shapes=[
            pltpu.VMEM((HD, N), jnp.bfloat16),
            pltpu.VMEM((N_DEV, SEG, s_chunk, Nh), jnp.bfloat16),
            pltpu.VMEM((N_DEV, SEG, s_chunk, Nh), jnp.bfloat16),
            pltpu.SemaphoreType.DMA((N_DEV - 1, SEG)),
            pltpu.SemaphoreType.DMA((N_DEV - 1, SEG)),
            pltpu.SemaphoreType.DMA((N_DEV - 1, SEG)),
            pltpu.SemaphoreType.DMA((N_DEV - 1, SEG)),
        ],
        compiler_params=pltpu.CompilerParams(collective_id=0),
    )(O_f, Wo)

    return out.reshape(B, s_chunk, N)
